# baseline (speedup 1.0000x reference)
import functools

import jax
import jax.numpy as jnp
from jax import lax
from jax.experimental import pallas as pl
from jax.experimental.pallas import tpu as pltpu
from jax.experimental.pallas import tpu_sc as plsc

_VOCAB = 100000
_D = 1024
_B = 4
_S = 2048
_N = _B * _S
_EPS = 1e-5

_NC = 2
_NS = 16
_NW = _NC * _NS
_NSLICE = 1
_SLICE = _N // _NSLICE
_PER_W = _SLICE // _NW
_CHUNK = 32
_NCHUNK = _PER_W // _CHUNK
_NBUF = 3


def _sc_gather_slice(ids, table, slice_idx):
    mesh = plsc.VectorSubcoreMesh(core_axis_name="c", subcore_axis_name="s")
    slice_off = slice_idx * _SLICE

    @functools.partial(
        pl.kernel,
        mesh=mesh,
        out_type=jax.ShapeDtypeStruct((_SLICE, _D), jnp.float32),
        scratch_types=[pltpu.VMEM((_PER_W,), jnp.int32)]
        + [pltpu.VMEM((_CHUNK, _D), jnp.float32)] * _NBUF
        + [pltpu.SemaphoreType.DMA] * (2 * _NBUF),
    )
    def gather_kernel(ids_hbm, table_hbm, out_hbm, idx_v, *scratch):
        bufs = scratch[:_NBUF]
        gsems = scratch[_NBUF : 2 * _NBUF]
        wsems = scratch[2 * _NBUF :]
        wid = lax.axis_index("s") * _NC + lax.axis_index("c")
        base = wid * _PER_W
        pltpu.sync_copy(ids_hbm.at[pl.ds(slice_off + base, _PER_W)], idx_v)

        def start_gather(c):
            b = c % _NBUF
            return pltpu.async_copy(
                table_hbm.at[idx_v.at[pl.ds(c * _CHUNK, _CHUNK)]], bufs[b], gsems[b]
            )

        gcopy = [None] * _NCHUNK
        wcopy = [None] * _NCHUNK
        gcopy[0] = start_gather(0)
        for c in range(_NCHUNK):
            b = c % _NBUF
            if c + 1 < _NCHUNK:
                if c + 1 >= _NBUF:
                    wcopy[c + 1 - _NBUF].wait()
                gcopy[c + 1] = start_gather(c + 1)
            gcopy[c].wait()
            wcopy[c] = pltpu.async_copy(
                bufs[b], out_hbm.at[pl.ds(base + c * _CHUNK, _CHUNK)], wsems[b]
            )
        for c in range(max(0, _NCHUNK - _NBUF), _NCHUNK):
            wcopy[c].wait()

    return gather_kernel(ids, table)


_LN_BLK = 2048
_BLK_PER_SLICE = _SLICE // _LN_BLK


def _ln_body_first(x_ref, g_ref, b_ref, o_ref):
    x = x_ref[...]
    ones = jnp.ones((_D, 8), jnp.float32)
    s1 = lax.dot_general(
        x, ones, (((1,), (0,)), ((), ())), preferred_element_type=jnp.float32
    )
    s2 = lax.dot_general(
        x * x, ones, (((1,), (0,)), ((), ())), preferred_element_type=jnp.float32
    )
    mu = s1[:, :1] * (1.0 / _D)
    var = s2[:, :1] * (1.0 / _D) - mu * mu
    o_ref[...] = (x - mu) * lax.rsqrt(var + _EPS) * g_ref[...] + b_ref[...]


def _ln_body_chain(x_ref, g_ref, b_ref, prev_ref, o_ref):
    del prev_ref
    _ln_body_first(x_ref, g_ref, b_ref, o_ref)


def _tc_layernorm_slice(x, gamma2d, beta2d, prev, slice_idx):
    off = slice_idx * _BLK_PER_SLICE
    out_spec = pl.BlockSpec((_LN_BLK, _D), lambda i: (i + off, 0))
    in_specs = [
        pl.BlockSpec((_LN_BLK, _D), lambda i: (i, 0)),
        pl.BlockSpec((1, _D), lambda i: (0, 0)),
        pl.BlockSpec((1, _D), lambda i: (0, 0)),
    ]
    out_shape = jax.ShapeDtypeStruct((_N, _D), jnp.float32)
    if prev is None:
        return pl.pallas_call(
            _ln_body_first,
            grid=(_BLK_PER_SLICE,),
            in_specs=in_specs,
            out_specs=out_spec,
            out_shape=out_shape,
        )(x, gamma2d, beta2d)
    return pl.pallas_call(
        _ln_body_chain,
        grid=(_BLK_PER_SLICE,),
        in_specs=in_specs + [pl.BlockSpec(memory_space=pltpu.HBM)],
        out_specs=out_spec,
        out_shape=out_shape,
        input_output_aliases={3: 0},
    )(x, gamma2d, beta2d, prev)


def kernel(tp_inputs, table, gamma, beta):
    ids = tp_inputs[..., 0].reshape(_N)
    mask = tp_inputs[..., 1].astype(jnp.float32)
    gamma2d = gamma.reshape(1, _D)
    beta2d = beta.reshape(1, _D)
    rows = [_sc_gather_slice(ids, table, k) for k in range(_NSLICE)]
    hidden = None
    for k in range(_NSLICE):
        hidden = _tc_layernorm_slice(rows[k], gamma2d, beta2d, hidden, k)
    return hidden.reshape(_B, _S, _D), mask


# speedup vs baseline: 1.0113x; 1.0113x over previous
import functools

import jax
import jax.numpy as jnp
from jax import lax
from jax.experimental import pallas as pl
from jax.experimental.pallas import tpu as pltpu
from jax.experimental.pallas import tpu_sc as plsc

_VOCAB = 100000
_D = 1024
_B = 4
_S = 2048
_N = _B * _S
_EPS = 1e-5

_NC = 2
_NS = 16
_NW = _NC * _NS
_SLICES = (5120, 3072)
_CHUNK = 32
_NBUF = 3


def _sc_gather_slice(ids, table, slice_off, slice_sz):
    mesh = plsc.VectorSubcoreMesh(core_axis_name="c", subcore_axis_name="s")
    per_w = slice_sz // _NW
    nchunk = per_w // _CHUNK

    @functools.partial(
        pl.kernel,
        mesh=mesh,
        out_type=jax.ShapeDtypeStruct((slice_sz, _D), jnp.float32),
        scratch_types=[pltpu.VMEM((per_w,), jnp.int32)]
        + [pltpu.VMEM((_CHUNK, _D), jnp.float32)] * _NBUF
        + [pltpu.SemaphoreType.DMA] * (2 * _NBUF),
    )
    def gather_kernel(ids_hbm, table_hbm, out_hbm, idx_v, *scratch):
        bufs = scratch[:_NBUF]
        gsems = scratch[_NBUF : 2 * _NBUF]
        wsems = scratch[2 * _NBUF :]
        wid = lax.axis_index("s") * _NC + lax.axis_index("c")
        base = wid * per_w
        pltpu.sync_copy(ids_hbm.at[pl.ds(slice_off + base, per_w)], idx_v)

        def start_gather(c):
            b = c % _NBUF
            return pltpu.async_copy(
                table_hbm.at[idx_v.at[pl.ds(c * _CHUNK, _CHUNK)]], bufs[b], gsems[b]
            )

        gcopy = [None] * nchunk
        wcopy = [None] * nchunk
        gcopy[0] = start_gather(0)
        for c in range(nchunk):
            b = c % _NBUF
            if c + 1 < nchunk:
                if c + 1 >= _NBUF:
                    wcopy[c + 1 - _NBUF].wait()
                gcopy[c + 1] = start_gather(c + 1)
            gcopy[c].wait()
            wcopy[c] = pltpu.async_copy(
                bufs[b], out_hbm.at[pl.ds(base + c * _CHUNK, _CHUNK)], wsems[b]
            )
        for c in range(max(0, nchunk - _NBUF), nchunk):
            wcopy[c].wait()

    return gather_kernel(ids, table)


_LN_BLK = 1024


def _ln_body_first(x_ref, g_ref, b_ref, o_ref):
    x = x_ref[...]
    mu = jnp.mean(x, axis=-1, keepdims=True)
    xc = x - mu
    var = jnp.mean(xc * xc, axis=-1, keepdims=True)
    o_ref[...] = xc * lax.rsqrt(var + _EPS) * g_ref[...] + b_ref[...]


def _ln_body_chain(x_ref, g_ref, b_ref, prev_ref, o_ref):
    del prev_ref
    _ln_body_first(x_ref, g_ref, b_ref, o_ref)


def _tc_layernorm_slice(x, gamma2d, beta2d, prev, blk_off):
    nblk = x.shape[0] // _LN_BLK
    out_spec = pl.BlockSpec((_LN_BLK, _D), lambda i: (i + blk_off, 0))
    in_specs = [
        pl.BlockSpec((_LN_BLK, _D), lambda i: (i, 0)),
        pl.BlockSpec((1, _D), lambda i: (0, 0)),
        pl.BlockSpec((1, _D), lambda i: (0, 0)),
    ]
    out_shape = jax.ShapeDtypeStruct((_N, _D), jnp.float32)
    if prev is None:
        return pl.pallas_call(
            _ln_body_first,
            grid=(nblk,),
            in_specs=in_specs,
            out_specs=out_spec,
            out_shape=out_shape,
        )(x, gamma2d, beta2d)
    return pl.pallas_call(
        _ln_body_chain,
        grid=(nblk,),
        in_specs=in_specs + [pl.BlockSpec(memory_space=pltpu.HBM)],
        out_specs=out_spec,
        out_shape=out_shape,
        input_output_aliases={3: 0},
    )(x, gamma2d, beta2d, prev)


def kernel(tp_inputs, table, gamma, beta):
    ids = tp_inputs[..., 0].reshape(_N)
    mask = tp_inputs[..., 1].astype(jnp.float32)
    gamma2d = gamma.reshape(1, _D)
    beta2d = beta.reshape(1, _D)
    offs = [0]
    for sz in _SLICES[:-1]:
        offs.append(offs[-1] + sz)
    rows = [
        _sc_gather_slice(ids, table, off, sz) for off, sz in zip(offs, _SLICES)
    ]
    hidden = None
    for r, off in zip(rows, offs):
        hidden = _tc_layernorm_slice(r, gamma2d, beta2d, hidden, off // _LN_BLK)
    return hidden.reshape(_B, _S, _D), mask


# speedup vs baseline: 1.0264x; 1.0150x over previous
import functools

import jax
import jax.numpy as jnp
from jax import lax
from jax.experimental import pallas as pl
from jax.experimental.pallas import tpu as pltpu
from jax.experimental.pallas import tpu_sc as plsc

_VOCAB = 100000
_D = 1024
_B = 4
_S = 2048
_N = _B * _S
_EPS = 1e-5

_NC = 2
_NS = 16
_NW = _NC * _NS
_SLICES = (8192,)
_CHUNK = 32
_NBUF = 3


def _sc_gather_slice(ids, table, slice_off, slice_sz):
    mesh = plsc.VectorSubcoreMesh(core_axis_name="c", subcore_axis_name="s")
    per_w = slice_sz // _NW
    nchunk = per_w // _CHUNK

    @functools.partial(
        pl.kernel,
        mesh=mesh,
        out_type=jax.ShapeDtypeStruct((slice_sz, _D), jnp.float32),
        scratch_types=[pltpu.VMEM((per_w,), jnp.int32)]
        + [pltpu.VMEM((_CHUNK, _D), jnp.float32)] * _NBUF
        + [pltpu.SemaphoreType.DMA] * (2 * _NBUF),
    )
    def gather_kernel(ids_hbm, table_hbm, out_hbm, idx_v, *scratch):
        bufs = scratch[:_NBUF]
        gsems = scratch[_NBUF : 2 * _NBUF]
        wsems = scratch[2 * _NBUF :]
        wid = lax.axis_index("s") * _NC + lax.axis_index("c")
        base = wid * per_w
        pltpu.sync_copy(ids_hbm.at[pl.ds(slice_off + base, per_w)], idx_v)

        def start_gather(c):
            b = c % _NBUF
            return pltpu.async_copy(
                table_hbm.at[idx_v.at[pl.ds(c * _CHUNK, _CHUNK)]], bufs[b], gsems[b]
            )

        gcopy = [None] * nchunk
        wcopy = [None] * nchunk
        gcopy[0] = start_gather(0)
        for c in range(nchunk):
            b = c % _NBUF
            if c + 1 < nchunk:
                if c + 1 >= _NBUF:
                    wcopy[c + 1 - _NBUF].wait()
                gcopy[c + 1] = start_gather(c + 1)
            gcopy[c].wait()
            wcopy[c] = pltpu.async_copy(
                bufs[b], out_hbm.at[pl.ds(base + c * _CHUNK, _CHUNK)], wsems[b]
            )
        for c in range(max(0, nchunk - _NBUF), nchunk):
            wcopy[c].wait()

    return gather_kernel(ids, table)


_LN_BLK = 2048


def _ln_body_first(x_ref, g_ref, b_ref, o_ref):
    x = x_ref[...]
    mu = jnp.mean(x, axis=-1, keepdims=True)
    xc = x - mu
    var = jnp.mean(xc * xc, axis=-1, keepdims=True)
    o_ref[...] = xc * lax.rsqrt(var + _EPS) * g_ref[...] + b_ref[...]


def _ln_body_chain(x_ref, g_ref, b_ref, prev_ref, o_ref):
    del prev_ref
    _ln_body_first(x_ref, g_ref, b_ref, o_ref)


def _tc_layernorm_slice(x, gamma2d, beta2d, prev, blk_off):
    nblk = x.shape[0] // _LN_BLK
    out_spec = pl.BlockSpec((_LN_BLK, _D), lambda i: (i + blk_off, 0))
    in_specs = [
        pl.BlockSpec((_LN_BLK, _D), lambda i: (i, 0)),
        pl.BlockSpec((1, _D), lambda i: (0, 0)),
        pl.BlockSpec((1, _D), lambda i: (0, 0)),
    ]
    out_shape = jax.ShapeDtypeStruct((_N, _D), jnp.float32)
    if prev is None:
        return pl.pallas_call(
            _ln_body_first,
            grid=(nblk,),
            in_specs=in_specs,
            out_specs=out_spec,
            out_shape=out_shape,
        )(x, gamma2d, beta2d)
    return pl.pallas_call(
        _ln_body_chain,
        grid=(nblk,),
        in_specs=in_specs + [pl.BlockSpec(memory_space=pltpu.HBM)],
        out_specs=out_spec,
        out_shape=out_shape,
        input_output_aliases={3: 0},
    )(x, gamma2d, beta2d, prev)


def kernel(tp_inputs, table, gamma, beta):
    ids = tp_inputs[..., 0].reshape(_N)
    mask = tp_inputs[..., 1].astype(jnp.float32)
    gamma2d = gamma.reshape(1, _D)
    beta2d = beta.reshape(1, _D)
    offs = [0]
    for sz in _SLICES[:-1]:
        offs.append(offs[-1] + sz)
    rows = [
        _sc_gather_slice(ids, table, off, sz) for off, sz in zip(offs, _SLICES)
    ]
    hidden = None
    for r, off in zip(rows, offs):
        hidden = _tc_layernorm_slice(r, gamma2d, beta2d, hidden, off // _LN_BLK)
    return hidden.reshape(_B, _S, _D), mask


# speedup vs baseline: 1.0627x; 1.0353x over previous
import functools

import jax
import jax.numpy as jnp
from jax import lax
from jax.experimental import pallas as pl
from jax.experimental.pallas import tpu as pltpu
from jax.experimental.pallas import tpu_sc as plsc

_VOCAB = 100000
_D = 1024
_B = 4
_S = 2048
_N = _B * _S
_EPS = 1e-5

_NC = 2
_NS = 16
_NW = _NC * _NS
_SLICES = (8192,)
_CHUNK = 32
_NBUF = 2


def _sc_gather_slice(ids, table, slice_off, slice_sz):
    mesh = plsc.VectorSubcoreMesh(core_axis_name="c", subcore_axis_name="s")
    per_w = slice_sz // _NW
    nchunk = per_w // _CHUNK

    @functools.partial(
        pl.kernel,
        mesh=mesh,
        out_type=jax.ShapeDtypeStruct((slice_sz, _D), jnp.float32),
        scratch_types=[pltpu.VMEM((per_w,), jnp.int32)]
        + [pltpu.VMEM((_CHUNK, _D), jnp.float32)] * _NBUF
        + [pltpu.SemaphoreType.DMA] * (2 * _NBUF),
    )
    def gather_kernel(ids_hbm, table_hbm, out_hbm, idx_v, *scratch):
        bufs = scratch[:_NBUF]
        gsems = scratch[_NBUF : 2 * _NBUF]
        wsems = scratch[2 * _NBUF :]
        wid = lax.axis_index("s") * _NC + lax.axis_index("c")
        base = wid * per_w
        pltpu.sync_copy(ids_hbm.at[pl.ds(slice_off + base, per_w)], idx_v)

        def start_gather(c):
            b = c % _NBUF
            return pltpu.async_copy(
                table_hbm.at[idx_v.at[pl.ds(c * _CHUNK, _CHUNK)]], bufs[b], gsems[b]
            )

        gcopy = [None] * nchunk
        wcopy = [None] * nchunk
        gcopy[0] = start_gather(0)
        for c in range(nchunk):
            b = c % _NBUF
            if c + 1 < nchunk:
                if c + 1 >= _NBUF:
                    wcopy[c + 1 - _NBUF].wait()
                gcopy[c + 1] = start_gather(c + 1)
            gcopy[c].wait()
            wcopy[c] = pltpu.async_copy(
                bufs[b], out_hbm.at[pl.ds(base + c * _CHUNK, _CHUNK)], wsems[b]
            )
        for c in range(max(0, nchunk - _NBUF), nchunk):
            wcopy[c].wait()

    return gather_kernel(ids, table)


_LN_BLK = 2048


def _ln_body_first(x_ref, g_ref, b_ref, o_ref):
    x = x_ref[...]
    mu = jnp.mean(x, axis=-1, keepdims=True)
    xc = x - mu
    var = jnp.mean(xc * xc, axis=-1, keepdims=True)
    o_ref[...] = xc * lax.rsqrt(var + _EPS) * g_ref[...] + b_ref[...]


def _ln_body_chain(x_ref, g_ref, b_ref, prev_ref, o_ref):
    del prev_ref
    _ln_body_first(x_ref, g_ref, b_ref, o_ref)


def _tc_layernorm_slice(x, gamma2d, beta2d, prev, blk_off):
    nblk = x.shape[0] // _LN_BLK
    out_spec = pl.BlockSpec((_LN_BLK, _D), lambda i: (i + blk_off, 0))
    in_specs = [
        pl.BlockSpec((_LN_BLK, _D), lambda i: (i, 0)),
        pl.BlockSpec((1, _D), lambda i: (0, 0)),
        pl.BlockSpec((1, _D), lambda i: (0, 0)),
    ]
    out_shape = jax.ShapeDtypeStruct((_N, _D), jnp.float32)
    if prev is None:
        return pl.pallas_call(
            _ln_body_first,
            grid=(nblk,),
            in_specs=in_specs,
            out_specs=out_spec,
            out_shape=out_shape,
        )(x, gamma2d, beta2d)
    return pl.pallas_call(
        _ln_body_chain,
        grid=(nblk,),
        in_specs=in_specs + [pl.BlockSpec(memory_space=pltpu.HBM)],
        out_specs=out_spec,
        out_shape=out_shape,
        input_output_aliases={3: 0},
    )(x, gamma2d, beta2d, prev)


def kernel(tp_inputs, table, gamma, beta):
    ids = tp_inputs[..., 0].reshape(_N)
    mask = tp_inputs[..., 1].astype(jnp.float32)
    gamma2d = gamma.reshape(1, _D)
    beta2d = beta.reshape(1, _D)
    offs = [0]
    for sz in _SLICES[:-1]:
        offs.append(offs[-1] + sz)
    rows = [
        _sc_gather_slice(ids, table, off, sz) for off, sz in zip(offs, _SLICES)
    ]
    hidden = None
    for r, off in zip(rows, offs):
        hidden = _tc_layernorm_slice(r, gamma2d, beta2d, hidden, off // _LN_BLK)
    return hidden.reshape(_B, _S, _D), mask
